# baseline (speedup 1.0000x reference)
import functools

import jax
import jax.numpy as jnp
from jax import lax
from jax.experimental import pallas as pl
from jax.experimental.pallas import tpu as pltpu
from jax.experimental.pallas import tpu_sc as plsc

N, K, E, O, I = 2048, 2, 64, 128, 128
M = O * I
NC, NS = 2, 16
NW = NC * NS
CW = M // NW
LG = 16
CH = 16
NCH = N // CH
LANES = 16


def _sc_weight_body(p0_hbm, p1_hbm, i0_hbm, i1_hbm, bankw_hbm, out_hbm,
                    bank_v, i0_v, i1_v, p0_v, p1_v, outb0, outb1,
                    sem_b0, sem_b1, sem_in):
    wid = lax.axis_index("s") * NC + lax.axis_index("c")
    col0 = wid * CW

    pltpu.async_copy(bankw_hbm.at[wid], bank_v, sem_in).wait()
    pltpu.async_copy(i0_hbm, i0_v, sem_in).wait()
    pltpu.async_copy(i1_hbm, i1_v, sem_in).wait()
    pltpu.async_copy(p0_hbm, p0_v, sem_in).wait()
    pltpu.async_copy(p1_hbm, p1_v, sem_in).wait()

    lanes = lax.iota(jnp.int32, LANES)
    zeros = jnp.zeros((LANES,), jnp.int32)

    def chunk(ch, outb, sem, first):
        t0 = ch * CH

        @pl.when(jnp.logical_not(first))
        def _():
            pltpu.make_async_copy(
                outb, out_hbm.at[pl.ds(t0, CH), pl.ds(col0, CW)], sem).wait()

        i0v = i0_v[pl.ds(t0, LG)]
        i1v = i1_v[pl.ds(t0, LG)]
        p0v = p0_v[pl.ds(t0, LG)]
        p1v = p1_v[pl.ds(t0, LG)]
        a0 = lax.shift_left(i0v, 9)
        b0 = lax.shift_left(i1v, 9)

        @plsc.parallel_loop(0, CW, 1, unroll=8, carry=(a0, b0, zeros))
        def _jloop(j, c):
            aa, bb, oo = c
            va = plsc.load_gather(bank_v, [aa])
            vb = plsc.load_gather(bank_v, [bb])
            plsc.store_scatter(outb, [lanes, oo], p0v * va + p1v * vb)
            return (aa + 1, bb + 1, oo + 1)

        pltpu.async_copy(
            outb, out_hbm.at[pl.ds(t0, CH), pl.ds(col0, CW)], sem)

    def pair(g, carry):
        chunk(g * 2, outb0, sem_b0, g == 0)
        chunk(g * 2 + 1, outb1, sem_b1, g == 0)
        return carry

    lax.fori_loop(0, NCH // 2, pair, 0, unroll=1)
    pltpu.make_async_copy(
        outb0, out_hbm.at[pl.ds(0, CH), pl.ds(col0, CW)], sem_b0).wait()
    pltpu.make_async_copy(
        outb1, out_hbm.at[pl.ds(0, CH), pl.ds(col0, CW)], sem_b1).wait()


_sc_weight = functools.partial(
    pl.kernel,
    out_type=jax.ShapeDtypeStruct((N, M), jnp.float32),
    mesh=plsc.VectorSubcoreMesh(core_axis_name="c", subcore_axis_name="s"),
    compiler_params=pltpu.CompilerParams(needs_layout_passes=False),
    scratch_types=[
        pltpu.VMEM((E * CW,), jnp.float32),
        pltpu.VMEM((N,), jnp.int32),
        pltpu.VMEM((N,), jnp.int32),
        pltpu.VMEM((N,), jnp.float32),
        pltpu.VMEM((N,), jnp.float32),
        pltpu.VMEM((CH, CW), jnp.float32),
        pltpu.VMEM((CH, CW), jnp.float32),
        pltpu.SemaphoreType.DMA,
        pltpu.SemaphoreType.DMA,
        pltpu.SemaphoreType.DMA,
    ],
)(_sc_weight_body)


def _tc_bias_kernel(bp_ref, bi_ref, bbank_ref, bout_ref):
    bp = bp_ref[...]
    bi = bi_ref[...]
    iota = lax.broadcasted_iota(jnp.int32, (N, E), 1)
    sb = (bp[:, 0:1] * (bi[:, 0:1] == iota).astype(jnp.float32)
          + bp[:, 1:2] * (bi[:, 1:2] == iota).astype(jnp.float32))
    bout_ref[...] = jnp.dot(sb, bbank_ref[...],
                            preferred_element_type=jnp.float32)


def kernel(weight_probs, weight_indices, bias_probs, bias_indices,
           weight_bank, bias_bank):
    wi = weight_indices.astype(jnp.int32)
    bi = bias_indices.astype(jnp.int32)
    bank2d = weight_bank.reshape(E, M)
    bankw = (bank2d.reshape(E, NW, CW).transpose(1, 0, 2).reshape(NW, E * CW))

    out2d = _sc_weight(weight_probs[:, 0], weight_probs[:, 1],
                       wi[:, 0], wi[:, 1], bankw)

    bout = pl.pallas_call(
        _tc_bias_kernel,
        out_shape=jax.ShapeDtypeStruct((N, O), jnp.float32),
    )(bias_probs, bi, bias_bank)

    return out2d.reshape(N, O, I), bout


# speedup vs baseline: 1.9455x; 1.9455x over previous
import functools

import jax
import jax.numpy as jnp
from jax import lax
from jax.experimental import pallas as pl
from jax.experimental.pallas import tpu as pltpu
from jax.experimental.pallas import tpu_sc as plsc

N, K, E, O, I = 2048, 2, 64, 128, 128
M = O * I
NC, NS = 2, 16
NW = NC * NS
CW = M // NW
LG = 16
CH = 16
CWP = CW + 1
NCH = N // CH
LANES = 16


def _sc_weight_body(p0_hbm, p1_hbm, i0_hbm, i1_hbm, bankw_hbm, out_hbm,
                    bank_v, i0_v, i1_v, p0_v, p1_v, outb0, outb1,
                    sem_b0, sem_b1, sem_in):
    wid = lax.axis_index("s") * NC + lax.axis_index("c")
    col0 = wid * CW

    pltpu.async_copy(bankw_hbm.at[wid], bank_v, sem_in).wait()
    pltpu.async_copy(i0_hbm, i0_v, sem_in).wait()
    pltpu.async_copy(i1_hbm, i1_v, sem_in).wait()
    pltpu.async_copy(p0_hbm, p0_v, sem_in).wait()
    pltpu.async_copy(p1_hbm, p1_v, sem_in).wait()

    lanes = lax.iota(jnp.int32, LANES)
    zeros = jnp.zeros((LANES,), jnp.int32)

    def chunk(ch, outb, sem, first):
        t0 = ch * CH

        @pl.when(jnp.logical_not(first))
        def _():
            pltpu.make_async_copy(
                outb.at[:, pl.ds(0, CW)],
                out_hbm.at[pl.ds(t0, CH), pl.ds(col0, CW)], sem).wait()

        i0v = i0_v[pl.ds(t0, LG)]
        i1v = i1_v[pl.ds(t0, LG)]
        p0v = p0_v[pl.ds(t0, LG)]
        p1v = p1_v[pl.ds(t0, LG)]
        a0 = i0v * CWP
        b0 = i1v * CWP

        @plsc.parallel_loop(0, CW, 1, unroll=8, carry=(a0, b0, zeros))
        def _jloop(j, c):
            aa, bb, oo = c
            va = plsc.load_gather(bank_v, [aa])
            vb = plsc.load_gather(bank_v, [bb])
            plsc.store_scatter(outb, [lanes, oo], p0v * va + p1v * vb)
            return (aa + 1, bb + 1, oo + 1)

        pltpu.async_copy(
            outb.at[:, pl.ds(0, CW)],
            out_hbm.at[pl.ds(t0, CH), pl.ds(col0, CW)], sem)

    def pair(g, carry):
        chunk(g * 2, outb0, sem_b0, g == 0)
        chunk(g * 2 + 1, outb1, sem_b1, g == 0)
        return carry

    lax.fori_loop(0, NCH // 2, pair, 0, unroll=1)
    pltpu.make_async_copy(
        outb0.at[:, pl.ds(0, CW)],
        out_hbm.at[pl.ds(0, CH), pl.ds(col0, CW)], sem_b0).wait()
    pltpu.make_async_copy(
        outb1.at[:, pl.ds(0, CW)],
        out_hbm.at[pl.ds(0, CH), pl.ds(col0, CW)], sem_b1).wait()


_sc_weight = functools.partial(
    pl.kernel,
    out_type=jax.ShapeDtypeStruct((N, M), jnp.float32),
    mesh=plsc.VectorSubcoreMesh(core_axis_name="c", subcore_axis_name="s"),
    compiler_params=pltpu.CompilerParams(needs_layout_passes=False),
    scratch_types=[
        pltpu.VMEM((E * CWP,), jnp.float32),
        pltpu.VMEM((N,), jnp.int32),
        pltpu.VMEM((N,), jnp.int32),
        pltpu.VMEM((N,), jnp.float32),
        pltpu.VMEM((N,), jnp.float32),
        pltpu.VMEM((CH, CWP), jnp.float32),
        pltpu.VMEM((CH, CWP), jnp.float32),
        pltpu.SemaphoreType.DMA,
        pltpu.SemaphoreType.DMA,
        pltpu.SemaphoreType.DMA,
    ],
)(_sc_weight_body)


def _tc_bias_kernel(bp_ref, bi_ref, bbank_ref, bout_ref):
    bp = bp_ref[...]
    bi = bi_ref[...]
    iota = lax.broadcasted_iota(jnp.int32, (N, E), 1)
    sb = (bp[:, 0:1] * (bi[:, 0:1] == iota).astype(jnp.float32)
          + bp[:, 1:2] * (bi[:, 1:2] == iota).astype(jnp.float32))
    bout_ref[...] = jnp.dot(sb, bbank_ref[...],
                            preferred_element_type=jnp.float32)


def kernel(weight_probs, weight_indices, bias_probs, bias_indices,
           weight_bank, bias_bank):
    wi = weight_indices.astype(jnp.int32)
    bi = bias_indices.astype(jnp.int32)
    bank2d = weight_bank.reshape(E, M)
    bankw = jnp.pad(bank2d.reshape(E, NW, CW).transpose(1, 0, 2),
                    ((0, 0), (0, 0), (0, CWP - CW))).reshape(NW, E * CWP)

    out2d = _sc_weight(weight_probs[:, 0], weight_probs[:, 1],
                       wi[:, 0], wi[:, 1], bankw)

    bout = pl.pallas_call(
        _tc_bias_kernel,
        out_shape=jax.ShapeDtypeStruct((N, O), jnp.float32),
    )(bias_probs, bi, bias_bank)

    return out2d.reshape(N, O, I), bout


# speedup vs baseline: 18.5642x; 9.5419x over previous
import jax
import jax.numpy as jnp
from jax.experimental import pallas as pl

N, K, E, O, I = 2048, 2, 64, 128, 128
M = O * I

TN = 128


def _mix_kernel(wp_ref, wi_ref, bp_ref, bi_ref, bank_ref, bbank_ref,
                out_ref, bout_ref):
    wp = wp_ref[...]
    wi = wi_ref[...]
    iota = jax.lax.broadcasted_iota(jnp.int32, (TN, E), 1)
    s = (wp[:, 0:1] * (wi[:, 0:1] == iota).astype(jnp.float32)
         + wp[:, 1:2] * (wi[:, 1:2] == iota).astype(jnp.float32))
    bank = bank_ref[...].reshape(E, M)
    res = jnp.dot(s, bank, preferred_element_type=jnp.float32)
    out_ref[...] = res.reshape(TN, O, I)

    bp = bp_ref[...]
    bi = bi_ref[...]
    sb = (bp[:, 0:1] * (bi[:, 0:1] == iota).astype(jnp.float32)
          + bp[:, 1:2] * (bi[:, 1:2] == iota).astype(jnp.float32))
    bout_ref[...] = jnp.dot(sb, bbank_ref[...],
                            preferred_element_type=jnp.float32)


def kernel(weight_probs, weight_indices, bias_probs, bias_indices,
           weight_bank, bias_bank):
    wi = weight_indices.astype(jnp.int32)
    bi = bias_indices.astype(jnp.int32)

    grid = (N // TN,)
    out, bout = pl.pallas_call(
        _mix_kernel,
        grid=grid,
        in_specs=[
            pl.BlockSpec((TN, K), lambda i: (i, 0)),
            pl.BlockSpec((TN, K), lambda i: (i, 0)),
            pl.BlockSpec((TN, K), lambda i: (i, 0)),
            pl.BlockSpec((TN, K), lambda i: (i, 0)),
            pl.BlockSpec((E, O, I), lambda i: (0, 0, 0)),
            pl.BlockSpec((E, O), lambda i: (0, 0)),
        ],
        out_specs=[
            pl.BlockSpec((TN, O, I), lambda i: (i, 0, 0)),
            pl.BlockSpec((TN, O), lambda i: (i, 0)),
        ],
        out_shape=[
            jax.ShapeDtypeStruct((N, O, I), jnp.float32),
            jax.ShapeDtypeStruct((N, O), jnp.float32),
        ],
    )(weight_probs, wi, bias_probs, bi, weight_bank, bias_bank)

    return out, bout


# speedup vs baseline: 19.4663x; 1.0486x over previous
import jax
import jax.numpy as jnp
from jax.experimental import pallas as pl

N, K, E, O, I = 2048, 2, 64, 128, 128
M = O * I

TN = 256


def _mix_kernel(wp_ref, wi_ref, bp_ref, bi_ref, bank_ref, bbank_ref,
                out_ref, bout_ref):
    wp = wp_ref[...]
    wi = wi_ref[...]
    iota = jax.lax.broadcasted_iota(jnp.int32, (TN, E), 1)
    s = (wp[:, 0:1] * (wi[:, 0:1] == iota).astype(jnp.float32)
         + wp[:, 1:2] * (wi[:, 1:2] == iota).astype(jnp.float32))
    bank = bank_ref[...].reshape(E, M)
    res = jnp.dot(s, bank, preferred_element_type=jnp.float32)
    out_ref[...] = res.reshape(TN, O, I)

    bp = bp_ref[...]
    bi = bi_ref[...]
    sb = (bp[:, 0:1] * (bi[:, 0:1] == iota).astype(jnp.float32)
          + bp[:, 1:2] * (bi[:, 1:2] == iota).astype(jnp.float32))
    bout_ref[...] = jnp.dot(sb, bbank_ref[...],
                            preferred_element_type=jnp.float32)


def kernel(weight_probs, weight_indices, bias_probs, bias_indices,
           weight_bank, bias_bank):
    wi = weight_indices.astype(jnp.int32)
    bi = bias_indices.astype(jnp.int32)

    grid = (N // TN,)
    out, bout = pl.pallas_call(
        _mix_kernel,
        grid=grid,
        in_specs=[
            pl.BlockSpec((TN, K), lambda i: (i, 0)),
            pl.BlockSpec((TN, K), lambda i: (i, 0)),
            pl.BlockSpec((TN, K), lambda i: (i, 0)),
            pl.BlockSpec((TN, K), lambda i: (i, 0)),
            pl.BlockSpec((E, O, I), lambda i: (0, 0, 0)),
            pl.BlockSpec((E, O), lambda i: (0, 0)),
        ],
        out_specs=[
            pl.BlockSpec((TN, O, I), lambda i: (i, 0, 0)),
            pl.BlockSpec((TN, O), lambda i: (i, 0)),
        ],
        out_shape=[
            jax.ShapeDtypeStruct((N, O, I), jnp.float32),
            jax.ShapeDtypeStruct((N, O), jnp.float32),
        ],
    )(weight_probs, wi, bias_probs, bi, weight_bank, bias_bank)

    return out, bout
